# baseline (speedup 1.0000x reference)
import functools

import jax
import jax.numpy as jnp
from jax import lax
from jax.experimental import pallas as pl
from jax.experimental.pallas import tpu as pltpu
from jax.experimental.pallas import tpu_sc as plsc

N = 10000
E = 320000
DIN = 128
DHID = 256
DOUT = 128

NC = 2
NS = 16
B = 128
E_TOT = E + N
G = 24
BLKS = 168
NGRP = BLKS // G
E_PAD = BLKS * B * NS
NROWS = 10240
STRIPE = NROWS // NS
HROWS = NROWS // B
HHALF = NROWS // 2
DSTRIPE = 1024
JUNK = 10008

_mesh = plsc.VectorSubcoreMesh(core_axis_name="c", subcore_axis_name="s")


def _deg_body(dst_hbm, deg_out, dst_g, ones_v, zero_v, acc_sh):
    c = lax.axis_index("c")
    s = lax.axis_index("s")

    def fill(i, x):
        ones_v[pl.ds(i * 16, 16)] = jnp.full((16,), 1.0, jnp.float32)
        return x

    lax.fori_loop(0, B // 16, fill, 0)

    def fillz(i, x):
        zero_v[pl.ds(i * 16, 16)] = jnp.zeros((16,), jnp.float32)
        return x

    lax.fori_loop(0, DSTRIPE // 16, fillz, 0)

    pltpu.sync_copy(zero_v, acc_sh.at[pl.ds(s * DSTRIPE, DSTRIPE)])
    plsc.subcore_barrier()

    def grp(g, x):
        pltpu.sync_copy(dst_hbm.at[s, pl.ds(g * G, G)], dst_g)

        def blk(b, y):
            pltpu.sync_copy(ones_v, acc_sh.at[dst_g.at[2 * b + c]], add=True)
            return y

        lax.fori_loop(0, G // 2, blk, 0)
        return x

    lax.fori_loop(0, NGRP, grp, 0)
    plsc.subcore_barrier()
    pltpu.sync_copy(acc_sh.at[pl.ds(s * DSTRIPE, DSTRIPE)],
                    deg_out.at[pl.ds((c * NS + s) * DSTRIPE, DSTRIPE)])


_deg_kernel = pl.kernel(
    _deg_body,
    out_type=jax.ShapeDtypeStruct((NC * NS * DSTRIPE,), jnp.float32),
    mesh=_mesh,
    scratch_types=[
        pltpu.VMEM((G, B), jnp.int32),
        pltpu.VMEM((B,), jnp.float32),
        pltpu.VMEM((DSTRIPE,), jnp.float32),
        pltpu.VMEM_SHARED((NS * DSTRIPE,), jnp.float32),
    ],
)


def _agg1_body(g_hbm, src_hbm, dst_hbm, acc_out,
               idx2_g, dst_g, buf0, buf1, acc_sh, semg0, semg1):
    c = lax.axis_index("c")
    s = lax.axis_index("s")
    Dh = DHID // 2
    bufs = (buf0, buf1)
    semg = (semg0, semg1)

    def fill_zero(i, x):
        buf0[i // (Dh // 16), pl.ds((i % (Dh // 16)) * 16, 16)] = (
            jnp.zeros((16,), jnp.float32))
        return x

    lax.fori_loop(0, B * Dh // 16, fill_zero, 0)

    def zero_stripe(i, x):
        pltpu.sync_copy(buf0, acc_sh.at[pl.ds(s * STRIPE + i * B, B)])
        return x

    lax.fori_loop(0, STRIPE // B, zero_stripe, 0)
    plsc.subcore_barrier()

    def grp(g, x):
        pltpu.sync_copy(src_hbm.at[s, pl.ds(g * G, G)], idx2_g)
        pltpu.sync_copy(dst_hbm.at[s, pl.ds(g * G, G)], dst_g)

        def fill_idx(i, y):
            r = i // (B // 16)
            k = i % (B // 16)
            v = idx2_g[r, pl.ds(k * 16, 16)]
            idx2_g[r, pl.ds(k * 16, 16)] = v + c * N
            return y

        lax.fori_loop(0, G * (B // 16), fill_idx, 0)

        descs = [None] * G
        descs[0] = pltpu.async_copy(g_hbm.at[idx2_g.at[0]], bufs[0], semg[0])
        for b in range(G):
            if b + 1 < G:
                descs[b + 1] = pltpu.async_copy(
                    g_hbm.at[idx2_g.at[b + 1]], bufs[(b + 1) % 2],
                    semg[(b + 1) % 2])
            descs[b].wait()
            pltpu.sync_copy(bufs[b % 2], acc_sh.at[dst_g.at[b]], add=True)
        return x

    lax.fori_loop(0, NGRP, grp, 0)
    plsc.subcore_barrier()
    pltpu.sync_copy(acc_sh.at[pl.ds(s * STRIPE, STRIPE)],
                    acc_out.at[c, pl.ds(s * STRIPE, STRIPE)])


_agg1 = pl.kernel(
    _agg1_body,
    out_type=jax.ShapeDtypeStruct((NC, NROWS, DHID // 2), jnp.float32),
    mesh=_mesh,
    scratch_types=[
        pltpu.VMEM((G, B), jnp.int32),
        pltpu.VMEM((G, B), jnp.int32),
        pltpu.VMEM((B, DHID // 2), jnp.float32),
        pltpu.VMEM((B, DHID // 2), jnp.float32),
        pltpu.VMEM_SHARED((NROWS, DHID // 2), jnp.float32),
        pltpu.SemaphoreType.DMA,
        pltpu.SemaphoreType.DMA,
    ],
)


def _agg2_body(g_hbm, src_hbm, dst_hbm, acc_out,
               src_g, dst_g, buf0, buf1, acc_sh, semg0, semg1):
    c = lax.axis_index("c")
    s = lax.axis_index("s")
    bufs = (buf0, buf1)
    semg = (semg0, semg1)

    def fill_zero(i, x):
        buf0[i // (DOUT // 16), pl.ds((i % (DOUT // 16)) * 16, 16)] = (
            jnp.zeros((16,), jnp.float32))
        return x

    lax.fori_loop(0, B * DOUT // 16, fill_zero, 0)

    def zero_stripe(i, x):
        pltpu.sync_copy(buf0, acc_sh.at[pl.ds(s * STRIPE + i * B, B)])
        return x

    lax.fori_loop(0, STRIPE // B, zero_stripe, 0)
    plsc.subcore_barrier()

    H = G // 2

    def grp(g, x):
        pltpu.sync_copy(src_hbm.at[s, pl.ds(g * G, G)], src_g)
        pltpu.sync_copy(dst_hbm.at[s, pl.ds(g * G, G)], dst_g)

        def fill_idx(i, y):
            r = i // (B // 16)
            k = i % (B // 16)
            v = src_g[r, pl.ds(k * 16, 16)]
            src_g[r, pl.ds(k * 16, 16)] = v + c * N
            return y

        lax.fori_loop(0, G * (B // 16), fill_idx, 0)

        descs = [None] * H
        descs[0] = pltpu.async_copy(g_hbm.at[src_g.at[c]], bufs[0], semg[0])
        for b in range(H):
            if b + 1 < H:
                descs[b + 1] = pltpu.async_copy(
                    g_hbm.at[src_g.at[2 * (b + 1) + c]], bufs[(b + 1) % 2],
                    semg[(b + 1) % 2])
            descs[b].wait()
            pltpu.sync_copy(bufs[b % 2], acc_sh.at[dst_g.at[2 * b + c]],
                            add=True)
        return x

    lax.fori_loop(0, NGRP, grp, 0)
    plsc.subcore_barrier()
    pltpu.sync_copy(acc_sh.at[pl.ds(s * STRIPE, STRIPE)],
                    acc_out.at[c, pl.ds(s * STRIPE, STRIPE)])


_agg2 = pl.kernel(
    _agg2_body,
    out_type=jax.ShapeDtypeStruct((NC, NROWS, DOUT), jnp.float32),
    mesh=_mesh,
    scratch_types=[
        pltpu.VMEM((G, B), jnp.int32),
        pltpu.VMEM((G, B), jnp.int32),
        pltpu.VMEM((B, DOUT), jnp.float32),
        pltpu.VMEM((B, DOUT), jnp.float32),
        pltpu.VMEM_SHARED((NROWS, DOUT), jnp.float32),
        pltpu.SemaphoreType.DMA,
        pltpu.SemaphoreType.DMA,
    ],
)


def _tc1_body(x_ref, w1_ref, degp_ref, g1_ref, dis_ref):
    deg = degp_ref[0, :] + degp_ref[1, :]
    dis = jnp.where(deg > 0, lax.rsqrt(deg), 0.0)
    dis_ref[...] = dis
    h = jnp.dot(x_ref[...], w1_ref[...], preferred_element_type=jnp.float32)
    g = h * dis[:N, None]
    g1_ref[0] = g[:, : DHID // 2]
    g1_ref[1] = g[:, DHID // 2:]


_tc1 = pl.pallas_call(
    _tc1_body,
    out_shape=(
        jax.ShapeDtypeStruct((2, N, DHID // 2), jnp.float32),
        jax.ShapeDtypeStruct((NROWS,), jnp.float32),
    ),
)


def _tc2_body(accp_ref, w2_ref, b1_ref, dis_ref, g2_ref):
    dis = dis_ref[...][:N]
    a = accp_ref[0, :N, :] * dis[:, None] + b1_ref[...][None, : DHID // 2]
    b = accp_ref[1, :N, :] * dis[:, None] + b1_ref[...][None, DHID // 2:]
    t = jnp.concatenate([jnp.maximum(a, 0.0), jnp.maximum(b, 0.0)], axis=1)
    g2 = jnp.dot(t, w2_ref[...], preferred_element_type=jnp.float32)
    g2 = g2 * dis[:, None]
    g2_ref[0] = g2
    g2_ref[1] = g2


_tc2 = pl.pallas_call(
    _tc2_body,
    out_shape=jax.ShapeDtypeStruct((2, N, DOUT), jnp.float32),
)


def _tc3_body(accp_ref, b2_ref, dis_ref, out_ref):
    dis = dis_ref[...][:N]
    out = accp_ref[0, :N, :] + accp_ref[1, :N, :]
    out_ref[...] = out * dis[:, None] + b2_ref[...][None, :]


_tc3 = pl.pallas_call(
    _tc3_body,
    out_shape=jax.ShapeDtypeStruct((N, DOUT), jnp.float32),
)


def kernel(x, edge_index, W1, b1, W2, b2):
    loop = jnp.arange(N, dtype=jnp.int32)
    pad = E_PAD - E_TOT
    src = jnp.concatenate(
        [edge_index[0], loop, jnp.zeros((pad,), jnp.int32)]).reshape(NS, BLKS, B)
    dst = jnp.concatenate(
        [edge_index[1], loop, jnp.full((pad,), JUNK, jnp.int32)]).reshape(NS, BLKS, B)

    degp = _deg_kernel(dst).reshape(NC, NS * DSTRIPE)[:, :NROWS]
    g1i, dis = _tc1(x, W1, degp)
    accp1 = _agg1(g1i.reshape(2 * N, DHID // 2), src, dst)
    g2 = _tc2(accp1, W2, b1, dis)
    accp2 = _agg2(g2.reshape(2 * N, DOUT), src, dst)
    return _tc3(accp2, b2, dis)


# speedup vs baseline: 1.0894x; 1.0894x over previous
import functools

import jax
import jax.numpy as jnp
from jax import lax
from jax.experimental import pallas as pl
from jax.experimental.pallas import tpu as pltpu
from jax.experimental.pallas import tpu_sc as plsc

N = 10000
E = 320000
DIN = 128
DHID = 256
DOUT = 128

NC = 2
NS = 16
B = 128
E_TOT = E + N
G = 24
BLKS = 168
NGRP = BLKS // G
E_PAD = BLKS * B * NS
NROWS = 10240
STRIPE = NROWS // NS
HROWS = NROWS // B
HHALF = NROWS // 2
DSTRIPE = 1024
JUNK = 10008

_mesh = plsc.VectorSubcoreMesh(core_axis_name="c", subcore_axis_name="s")


def _deg_body(dst_hbm, deg_out, dst_g, ones_v, zero_v, acc_sh):
    c = lax.axis_index("c")
    s = lax.axis_index("s")

    def fill(i, x):
        ones_v[pl.ds(i * 16, 16)] = jnp.full((16,), 1.0, jnp.float32)
        return x

    lax.fori_loop(0, B // 16, fill, 0)

    def fillz(i, x):
        zero_v[pl.ds(i * 16, 16)] = jnp.zeros((16,), jnp.float32)
        return x

    lax.fori_loop(0, DSTRIPE // 16, fillz, 0)

    pltpu.sync_copy(zero_v, acc_sh.at[pl.ds(s * DSTRIPE, DSTRIPE)])
    plsc.subcore_barrier()

    def grp(g, x):
        pltpu.sync_copy(dst_hbm.at[s, pl.ds(g * G, G)], dst_g)

        def blk(b, y):
            pltpu.sync_copy(ones_v, acc_sh.at[dst_g.at[2 * b + c]], add=True)
            return y

        lax.fori_loop(0, G // 2, blk, 0)
        return x

    lax.fori_loop(0, NGRP, grp, 0)
    plsc.subcore_barrier()
    pltpu.sync_copy(acc_sh.at[pl.ds(s * DSTRIPE, DSTRIPE)],
                    deg_out.at[pl.ds((c * NS + s) * DSTRIPE, DSTRIPE)])


_deg_kernel = pl.kernel(
    _deg_body,
    out_type=jax.ShapeDtypeStruct((NC * NS * DSTRIPE,), jnp.float32),
    mesh=_mesh,
    scratch_types=[
        pltpu.VMEM((G, B), jnp.int32),
        pltpu.VMEM((B,), jnp.float32),
        pltpu.VMEM((DSTRIPE,), jnp.float32),
        pltpu.VMEM_SHARED((NS * DSTRIPE,), jnp.float32),
    ],
)


def _agg1_body(g_hbm, src_hbm, dst_hbm, acc_out,
               idx2_g, dst_g, buf0, buf1, acc_sh, semg0, semg1):
    c = lax.axis_index("c")
    s = lax.axis_index("s")
    Dh = DHID // 2
    bufs = (buf0, buf1)
    semg = (semg0, semg1)

    def fill_zero(i, x):
        buf0[i // (Dh // 16), pl.ds((i % (Dh // 16)) * 16, 16)] = (
            jnp.zeros((16,), jnp.float32))
        return x

    lax.fori_loop(0, B * Dh // 16, fill_zero, 0)

    def zero_stripe(i, x):
        pltpu.sync_copy(buf0, acc_sh.at[pl.ds(s * STRIPE + i * B, B)])
        return x

    lax.fori_loop(0, STRIPE // B, zero_stripe, 0)
    plsc.subcore_barrier()

    def grp(g, x):
        pltpu.sync_copy(src_hbm.at[s, pl.ds(g * G, G)], idx2_g)
        pltpu.sync_copy(dst_hbm.at[s, pl.ds(g * G, G)], dst_g)

        def fill_idx(i, y):
            r = i // (B // 16)
            k = i % (B // 16)
            v = idx2_g[r, pl.ds(k * 16, 16)]
            idx2_g[r, pl.ds(k * 16, 16)] = v + c * N
            return y

        lax.fori_loop(0, G * (B // 16), fill_idx, 0)

        descs = [None] * G
        descs[0] = pltpu.async_copy(g_hbm.at[idx2_g.at[0]], bufs[0], semg[0])
        for b in range(G):
            if b + 1 < G:
                descs[b + 1] = pltpu.async_copy(
                    g_hbm.at[idx2_g.at[b + 1]], bufs[(b + 1) % 2],
                    semg[(b + 1) % 2])
            descs[b].wait()
            pltpu.sync_copy(bufs[b % 2], acc_sh.at[dst_g.at[b]], add=True)
        return x

    lax.fori_loop(0, NGRP, grp, 0)
    plsc.subcore_barrier()
    pltpu.sync_copy(acc_sh.at[pl.ds(s * STRIPE, STRIPE)],
                    acc_out.at[c, pl.ds(s * STRIPE, STRIPE)])


_agg1 = pl.kernel(
    _agg1_body,
    out_type=jax.ShapeDtypeStruct((NC, NROWS, DHID // 2), jnp.float32),
    mesh=_mesh,
    scratch_types=[
        pltpu.VMEM((G, B), jnp.int32),
        pltpu.VMEM((G, B), jnp.int32),
        pltpu.VMEM((B, DHID // 2), jnp.float32),
        pltpu.VMEM((B, DHID // 2), jnp.float32),
        pltpu.VMEM_SHARED((NROWS, DHID // 2), jnp.float32),
        pltpu.SemaphoreType.DMA,
        pltpu.SemaphoreType.DMA,
    ],
)


def _agg2_body(g_hbm, src_hbm, dst_hbm, acc_out,
               src_g, dst_g, buf0, buf1, acc_sh, semg0, semg1):
    c = lax.axis_index("c")
    s = lax.axis_index("s")
    bufs = (buf0, buf1)
    semg = (semg0, semg1)

    def fill_zero(i, x):
        buf0[i // (DOUT // 16), pl.ds((i % (DOUT // 16)) * 16, 16)] = (
            jnp.zeros((16,), jnp.float32))
        return x

    lax.fori_loop(0, B * DOUT // 16, fill_zero, 0)

    def zero_stripe(i, x):
        pltpu.sync_copy(buf0, acc_sh.at[pl.ds(s * STRIPE + i * B, B)])
        return x

    lax.fori_loop(0, STRIPE // B, zero_stripe, 0)
    plsc.subcore_barrier()

    H = G // 2

    def grp(g, x):
        pltpu.sync_copy(src_hbm.at[s, pl.ds(g * G, G)], src_g)
        pltpu.sync_copy(dst_hbm.at[s, pl.ds(g * G, G)], dst_g)

        descs = [None] * H
        descs[0] = pltpu.async_copy(g_hbm.at[src_g.at[c]], bufs[0], semg[0])
        for b in range(H):
            if b + 1 < H:
                descs[b + 1] = pltpu.async_copy(
                    g_hbm.at[src_g.at[2 * (b + 1) + c]], bufs[(b + 1) % 2],
                    semg[(b + 1) % 2])
            descs[b].wait()
            pltpu.sync_copy(bufs[b % 2], acc_sh.at[dst_g.at[2 * b + c]],
                            add=True)
        return x

    lax.fori_loop(0, NGRP, grp, 0)
    plsc.subcore_barrier()
    pltpu.sync_copy(acc_sh.at[pl.ds(s * STRIPE, STRIPE)],
                    acc_out.at[c, pl.ds(s * STRIPE, STRIPE)])


_agg2 = pl.kernel(
    _agg2_body,
    out_type=jax.ShapeDtypeStruct((NC, NROWS, DOUT), jnp.float32),
    mesh=_mesh,
    scratch_types=[
        pltpu.VMEM((G, B), jnp.int32),
        pltpu.VMEM((G, B), jnp.int32),
        pltpu.VMEM((B, DOUT), jnp.float32),
        pltpu.VMEM((B, DOUT), jnp.float32),
        pltpu.VMEM_SHARED((NROWS, DOUT), jnp.float32),
        pltpu.SemaphoreType.DMA,
        pltpu.SemaphoreType.DMA,
    ],
)


def _tc1_body(x_ref, w1_ref, degp_ref, g1_ref, dis_ref):
    deg = degp_ref[0, :] + degp_ref[1, :]
    dis = jnp.where(deg > 0, lax.rsqrt(deg), 0.0)
    dis_ref[...] = dis
    h = jnp.dot(x_ref[...], w1_ref[...], preferred_element_type=jnp.float32)
    g = h * dis[:N, None]
    g1_ref[0] = g[:, : DHID // 2]
    g1_ref[1] = g[:, DHID // 2:]


_tc1 = pl.pallas_call(
    _tc1_body,
    out_shape=(
        jax.ShapeDtypeStruct((2, N, DHID // 2), jnp.float32),
        jax.ShapeDtypeStruct((NROWS,), jnp.float32),
    ),
)


def _tc2_body(accp_ref, w2_ref, b1_ref, dis_ref, g2_ref):
    dis = dis_ref[...][:N]
    a = accp_ref[0, :N, :] * dis[:, None] + b1_ref[...][None, : DHID // 2]
    b = accp_ref[1, :N, :] * dis[:, None] + b1_ref[...][None, DHID // 2:]
    t = jnp.concatenate([jnp.maximum(a, 0.0), jnp.maximum(b, 0.0)], axis=1)
    g2 = jnp.dot(t, w2_ref[...], preferred_element_type=jnp.float32)
    g2_ref[...] = g2 * dis[:, None]


_tc2 = pl.pallas_call(
    _tc2_body,
    out_shape=jax.ShapeDtypeStruct((N, DOUT), jnp.float32),
)


def _tc3_body(accp_ref, b2_ref, dis_ref, out_ref):
    dis = dis_ref[...][:N]
    out = accp_ref[0, :N, :] + accp_ref[1, :N, :]
    out_ref[...] = out * dis[:, None] + b2_ref[...][None, :]


_tc3 = pl.pallas_call(
    _tc3_body,
    out_shape=jax.ShapeDtypeStruct((N, DOUT), jnp.float32),
)


def kernel(x, edge_index, W1, b1, W2, b2):
    loop = jnp.arange(N, dtype=jnp.int32)
    pad = E_PAD - E_TOT
    src = jnp.concatenate(
        [edge_index[0], loop, jnp.zeros((pad,), jnp.int32)]).reshape(NS, BLKS, B)
    dst = jnp.concatenate(
        [edge_index[1], loop, jnp.full((pad,), JUNK, jnp.int32)]).reshape(NS, BLKS, B)

    degp = _deg_kernel(dst).reshape(NC, NS * DSTRIPE)[:, :NROWS]
    g1i, dis = _tc1(x, W1, degp)
    accp1 = _agg1(g1i.reshape(2 * N, DHID // 2), src, dst)
    g2 = _tc2(accp1, W2, b1, dis)
    accp2 = _agg2(g2, src, dst)
    return _tc3(accp2, b2, dis)


# speedup vs baseline: 1.2918x; 1.1859x over previous
import functools

import jax
import jax.numpy as jnp
from jax import lax
from jax.experimental import pallas as pl
from jax.experimental.pallas import tpu as pltpu
from jax.experimental.pallas import tpu_sc as plsc

N = 10000
E = 320000
DIN = 128
DHID = 256
DOUT = 128

NC = 2
NS = 16
B = 128
E_TOT = E + N
G = 24
BLKS = 168
NGRP = BLKS // G
E_PAD = BLKS * B * NS
NROWS = 10240
STRIPE = NROWS // NS
HROWS = NROWS // B
HHALF = NROWS // 2
DSTRIPE = 1024
JUNK = 10008

_mesh = plsc.VectorSubcoreMesh(core_axis_name="c", subcore_axis_name="s")


def _deg_body(dst_hbm, deg_out, dst_g, ones_v, zero_v, acc_sh):
    c = lax.axis_index("c")
    s = lax.axis_index("s")

    def fill(i, x):
        ones_v[pl.ds(i * 16, 16)] = jnp.full((16,), 1.0, jnp.float32)
        return x

    lax.fori_loop(0, B // 16, fill, 0)

    def fillz(i, x):
        zero_v[pl.ds(i * 16, 16)] = jnp.zeros((16,), jnp.float32)
        return x

    lax.fori_loop(0, DSTRIPE // 16, fillz, 0)

    pltpu.sync_copy(zero_v, acc_sh.at[pl.ds(s * DSTRIPE, DSTRIPE)])
    plsc.subcore_barrier()

    def grp(g, x):
        pltpu.sync_copy(dst_hbm.at[s, pl.ds(g * G, G)], dst_g)

        def blk(b, y):
            pltpu.sync_copy(ones_v, acc_sh.at[dst_g.at[2 * b + c]], add=True)
            return y

        lax.fori_loop(0, G // 2, blk, 0)
        return x

    lax.fori_loop(0, NGRP, grp, 0)
    plsc.subcore_barrier()
    pltpu.sync_copy(acc_sh.at[pl.ds(s * DSTRIPE, DSTRIPE)],
                    deg_out.at[pl.ds((c * NS + s) * DSTRIPE, DSTRIPE)])


_deg_kernel = pl.kernel(
    _deg_body,
    out_type=jax.ShapeDtypeStruct((NC * NS * DSTRIPE,), jnp.float32),
    mesh=_mesh,
    scratch_types=[
        pltpu.VMEM((G, B), jnp.int32),
        pltpu.VMEM((B,), jnp.float32),
        pltpu.VMEM((DSTRIPE,), jnp.float32),
        pltpu.VMEM_SHARED((NS * DSTRIPE,), jnp.float32),
    ],
)


def _agg1_body(g_hbm, src_hbm, dst_hbm, acc_out,
               idx2_g, dst_g, buf0, buf1, acc_sh, semg0, semg1):
    c = lax.axis_index("c")
    s = lax.axis_index("s")
    Dh = DHID // 2
    bufs = (buf0, buf1)
    semg = (semg0, semg1)

    def fill_zero(i, x):
        buf0[i // (Dh // 16), pl.ds((i % (Dh // 16)) * 16, 16)] = (
            jnp.zeros((16,), jnp.float32))
        return x

    lax.fori_loop(0, B * Dh // 16, fill_zero, 0)

    def zero_stripe(i, x):
        pltpu.sync_copy(buf0, acc_sh.at[pl.ds(s * STRIPE + i * B, B)])
        return x

    lax.fori_loop(0, STRIPE // B, zero_stripe, 0)
    plsc.subcore_barrier()

    def grp(g, x):
        pltpu.sync_copy(src_hbm.at[s, pl.ds(g * G, G)], idx2_g)
        pltpu.sync_copy(dst_hbm.at[s, pl.ds(g * G, G)], dst_g)

        def fill_idx(i, y):
            r = i // (B // 16)
            k = i % (B // 16)
            v = idx2_g[r, pl.ds(k * 16, 16)]
            idx2_g[r, pl.ds(k * 16, 16)] = v + c * N
            return y

        lax.fori_loop(0, G * (B // 16), fill_idx, 0)

        descs = [None] * G
        descs[0] = pltpu.async_copy(g_hbm.at[idx2_g.at[0]], bufs[0], semg[0])
        for b in range(G):
            if b + 1 < G:
                descs[b + 1] = pltpu.async_copy(
                    g_hbm.at[idx2_g.at[b + 1]], bufs[(b + 1) % 2],
                    semg[(b + 1) % 2])
            descs[b].wait()
            pltpu.sync_copy(bufs[b % 2], acc_sh.at[dst_g.at[b]], add=True)
        return x

    lax.fori_loop(0, NGRP, grp, 0)
    plsc.subcore_barrier()
    pltpu.sync_copy(acc_sh.at[pl.ds(s * STRIPE, STRIPE)],
                    acc_out.at[c, pl.ds(s * STRIPE, STRIPE)])


_agg1 = pl.kernel(
    _agg1_body,
    out_type=jax.ShapeDtypeStruct((NC, NROWS, DHID // 2), jnp.float32),
    mesh=_mesh,
    scratch_types=[
        pltpu.VMEM((G, B), jnp.int32),
        pltpu.VMEM((G, B), jnp.int32),
        pltpu.VMEM((B, DHID // 2), jnp.float32),
        pltpu.VMEM((B, DHID // 2), jnp.float32),
        pltpu.VMEM_SHARED((NROWS, DHID // 2), jnp.float32),
        pltpu.SemaphoreType.DMA,
        pltpu.SemaphoreType.DMA,
    ],
)


def _agg2_body(g_hbm, src_hbm, dst_hbm, acc_out,
               src_g, dst_g, buf0, buf1, acc_sh, semg0, semg1):
    c = lax.axis_index("c")
    s = lax.axis_index("s")
    bufs = (buf0, buf1)
    semg = (semg0, semg1)

    def fill_zero(i, x):
        buf0[i // (DOUT // 16), pl.ds((i % (DOUT // 16)) * 16, 16)] = (
            jnp.zeros((16,), jnp.float32))
        return x

    lax.fori_loop(0, B * DOUT // 16, fill_zero, 0)

    def zero_stripe(i, x):
        pltpu.sync_copy(buf0, acc_sh.at[pl.ds(s * STRIPE + i * B, B)])
        return x

    lax.fori_loop(0, STRIPE // B, zero_stripe, 0)
    plsc.subcore_barrier()

    H = G // 2

    def grp(g, x):
        pltpu.sync_copy(src_hbm.at[s, pl.ds(g * G, G)], src_g)
        pltpu.sync_copy(dst_hbm.at[s, pl.ds(g * G, G)], dst_g)

        def fill_idx(i, y):
            r = i // (B // 16)
            k = i % (B // 16)
            v = src_g[r, pl.ds(k * 16, 16)]
            src_g[r, pl.ds(k * 16, 16)] = v + c * N
            return y

        lax.fori_loop(0, G * (B // 16), fill_idx, 0)

        descs = [None] * H
        descs[0] = pltpu.async_copy(g_hbm.at[src_g.at[c]], bufs[0], semg[0])
        for b in range(H):
            if b + 1 < H:
                descs[b + 1] = pltpu.async_copy(
                    g_hbm.at[src_g.at[2 * (b + 1) + c]], bufs[(b + 1) % 2],
                    semg[(b + 1) % 2])
            descs[b].wait()
            pltpu.sync_copy(bufs[b % 2], acc_sh.at[dst_g.at[2 * b + c]],
                            add=True)
        return x

    lax.fori_loop(0, NGRP, grp, 0)
    plsc.subcore_barrier()
    pltpu.sync_copy(acc_sh.at[pl.ds(s * STRIPE, STRIPE)],
                    acc_out.at[c, pl.ds(s * STRIPE, STRIPE)])


_agg2 = pl.kernel(
    _agg2_body,
    out_type=jax.ShapeDtypeStruct((NC, NROWS, DOUT), jnp.float32),
    mesh=_mesh,
    scratch_types=[
        pltpu.VMEM((G, B), jnp.int32),
        pltpu.VMEM((G, B), jnp.int32),
        pltpu.VMEM((B, DOUT), jnp.float32),
        pltpu.VMEM((B, DOUT), jnp.float32),
        pltpu.VMEM_SHARED((NROWS, DOUT), jnp.float32),
        pltpu.SemaphoreType.DMA,
        pltpu.SemaphoreType.DMA,
    ],
)


def _tc1_body(x_ref, w1_ref, degp_ref, g1_ref, dis_ref):
    deg = degp_ref[0, :] + degp_ref[1, :]
    dis = jnp.where(deg > 0, lax.rsqrt(deg), 0.0)
    dis_ref[...] = dis
    h = jnp.dot(x_ref[...], w1_ref[...], preferred_element_type=jnp.float32)
    g = h * dis[:N, None]
    g1_ref[0] = g[:, : DHID // 2]
    g1_ref[1] = g[:, DHID // 2:]


_tc1 = pl.pallas_call(
    _tc1_body,
    out_shape=(
        jax.ShapeDtypeStruct((2, N, DHID // 2), jnp.float32),
        jax.ShapeDtypeStruct((NROWS,), jnp.float32),
    ),
)


def _tc2_body(accp_ref, w2_ref, b1_ref, dis_ref, g2_ref):
    dis = dis_ref[...][:N]
    a = accp_ref[0, :N, :] * dis[:, None] + b1_ref[...][None, : DHID // 2]
    b = accp_ref[1, :N, :] * dis[:, None] + b1_ref[...][None, DHID // 2:]
    t = jnp.concatenate([jnp.maximum(a, 0.0), jnp.maximum(b, 0.0)], axis=1)
    g2 = jnp.dot(t, w2_ref[...], preferred_element_type=jnp.float32)
    g2_ref[...] = g2 * dis[:, None]


_tc2 = pl.pallas_call(
    _tc2_body,
    out_shape=jax.ShapeDtypeStruct((N, DOUT), jnp.float32),
)


def _tc3_body(accp_ref, b2_ref, dis_ref, out_ref):
    dis = dis_ref[...][:N]
    out = accp_ref[0, :N, :] + accp_ref[1, :N, :]
    out_ref[...] = out * dis[:, None] + b2_ref[...][None, :]


_tc3 = pl.pallas_call(
    _tc3_body,
    out_shape=jax.ShapeDtypeStruct((N, DOUT), jnp.float32),
)


def kernel(x, edge_index, W1, b1, W2, b2):
    loop = jnp.arange(N, dtype=jnp.int32)
    pad = E_PAD - E_TOT
    src = jnp.concatenate(
        [edge_index[0], loop, jnp.zeros((pad,), jnp.int32)]).reshape(NS, BLKS, B)
    dst = jnp.concatenate(
        [edge_index[1], loop, jnp.full((pad,), JUNK, jnp.int32)]).reshape(NS, BLKS, B)

    degp = _deg_kernel(dst).reshape(NC, NS * DSTRIPE)[:, :NROWS]
    g1i, dis = _tc1(x, W1, degp)
    accp1 = _agg1(g1i.reshape(2 * N, DHID // 2), src, dst)
    g2 = _tc2(accp1, W2, b1, dis)
    accp2 = _agg2(jnp.concatenate([g2, g2], axis=0), src, dst)
    return _tc3(accp2, b2, dis)


# speedup vs baseline: 1.3166x; 1.0192x over previous
import functools

import jax
import jax.numpy as jnp
from jax import lax
from jax.experimental import pallas as pl
from jax.experimental.pallas import tpu as pltpu
from jax.experimental.pallas import tpu_sc as plsc

N = 10000
E = 320000
DIN = 128
DHID = 256
DOUT = 128

NC = 2
NS = 16
B = 128
E_TOT = E + N
G = 56
BLKS = 168
NGRP = BLKS // G
E_PAD = BLKS * B * NS
NROWS = 10240
STRIPE = NROWS // NS
HROWS = NROWS // B
HHALF = NROWS // 2
DSTRIPE = 1024
JUNK = 10008

_mesh = plsc.VectorSubcoreMesh(core_axis_name="c", subcore_axis_name="s")


def _deg_body(dst_hbm, deg_out, dst_g, ones_v, zero_v, acc_sh):
    c = lax.axis_index("c")
    s = lax.axis_index("s")

    def fill(i, x):
        ones_v[pl.ds(i * 16, 16)] = jnp.full((16,), 1.0, jnp.float32)
        return x

    lax.fori_loop(0, B // 16, fill, 0)

    def fillz(i, x):
        zero_v[pl.ds(i * 16, 16)] = jnp.zeros((16,), jnp.float32)
        return x

    lax.fori_loop(0, DSTRIPE // 16, fillz, 0)

    pltpu.sync_copy(zero_v, acc_sh.at[pl.ds(s * DSTRIPE, DSTRIPE)])
    plsc.subcore_barrier()

    def grp(g, x):
        pltpu.sync_copy(dst_hbm.at[s, pl.ds(g * G, G)], dst_g)

        def blk(b, y):
            pltpu.sync_copy(ones_v, acc_sh.at[dst_g.at[2 * b + c]], add=True)
            return y

        lax.fori_loop(0, G // 2, blk, 0)
        return x

    lax.fori_loop(0, NGRP, grp, 0)
    plsc.subcore_barrier()
    pltpu.sync_copy(acc_sh.at[pl.ds(s * DSTRIPE, DSTRIPE)],
                    deg_out.at[pl.ds((c * NS + s) * DSTRIPE, DSTRIPE)])


_deg_kernel = pl.kernel(
    _deg_body,
    out_type=jax.ShapeDtypeStruct((NC * NS * DSTRIPE,), jnp.float32),
    mesh=_mesh,
    scratch_types=[
        pltpu.VMEM((G, B), jnp.int32),
        pltpu.VMEM((B,), jnp.float32),
        pltpu.VMEM((DSTRIPE,), jnp.float32),
        pltpu.VMEM_SHARED((NS * DSTRIPE,), jnp.float32),
    ],
)


def _agg1_body(g_hbm, src_hbm, dst_hbm, acc_out,
               idx2_g, dst_g, buf0, buf1, acc_sh, semg0, semg1):
    c = lax.axis_index("c")
    s = lax.axis_index("s")
    Dh = DHID // 2
    bufs = (buf0, buf1)
    semg = (semg0, semg1)

    def fill_zero(i, x):
        buf0[i // (Dh // 16), pl.ds((i % (Dh // 16)) * 16, 16)] = (
            jnp.zeros((16,), jnp.float32))
        return x

    lax.fori_loop(0, B * Dh // 16, fill_zero, 0)

    def zero_stripe(i, x):
        pltpu.sync_copy(buf0, acc_sh.at[pl.ds(s * STRIPE + i * B, B)])
        return x

    lax.fori_loop(0, STRIPE // B, zero_stripe, 0)
    plsc.subcore_barrier()

    def grp(g, x):
        pltpu.sync_copy(src_hbm.at[s, pl.ds(g * G, G)], idx2_g)
        pltpu.sync_copy(dst_hbm.at[s, pl.ds(g * G, G)], dst_g)

        def fill_idx(i, y):
            r = i // (B // 16)
            k = i % (B // 16)
            v = idx2_g[r, pl.ds(k * 16, 16)]
            idx2_g[r, pl.ds(k * 16, 16)] = v + c * N
            return y

        lax.fori_loop(0, G * (B // 16), fill_idx, 0)

        descs = [None] * G
        descs[0] = pltpu.async_copy(g_hbm.at[idx2_g.at[0]], bufs[0], semg[0])
        for b in range(G):
            if b + 1 < G:
                descs[b + 1] = pltpu.async_copy(
                    g_hbm.at[idx2_g.at[b + 1]], bufs[(b + 1) % 2],
                    semg[(b + 1) % 2])
            descs[b].wait()
            pltpu.sync_copy(bufs[b % 2], acc_sh.at[dst_g.at[b]], add=True)
        return x

    lax.fori_loop(0, NGRP, grp, 0)
    plsc.subcore_barrier()
    pltpu.sync_copy(acc_sh.at[pl.ds(s * STRIPE, STRIPE)],
                    acc_out.at[c, pl.ds(s * STRIPE, STRIPE)])


_agg1 = pl.kernel(
    _agg1_body,
    out_type=jax.ShapeDtypeStruct((NC, NROWS, DHID // 2), jnp.float32),
    mesh=_mesh,
    scratch_types=[
        pltpu.VMEM((G, B), jnp.int32),
        pltpu.VMEM((G, B), jnp.int32),
        pltpu.VMEM((B, DHID // 2), jnp.float32),
        pltpu.VMEM((B, DHID // 2), jnp.float32),
        pltpu.VMEM_SHARED((NROWS, DHID // 2), jnp.float32),
        pltpu.SemaphoreType.DMA,
        pltpu.SemaphoreType.DMA,
    ],
)


def _agg2_body(g_hbm, src_hbm, dst_hbm, acc_out,
               src_g, dst_g, buf0, buf1, acc_sh, semg0, semg1):
    c = lax.axis_index("c")
    s = lax.axis_index("s")
    bufs = (buf0, buf1)
    semg = (semg0, semg1)

    def fill_zero(i, x):
        buf0[i // (DOUT // 16), pl.ds((i % (DOUT // 16)) * 16, 16)] = (
            jnp.zeros((16,), jnp.float32))
        return x

    lax.fori_loop(0, B * DOUT // 16, fill_zero, 0)

    def zero_stripe(i, x):
        pltpu.sync_copy(buf0, acc_sh.at[pl.ds(s * STRIPE + i * B, B)])
        return x

    lax.fori_loop(0, STRIPE // B, zero_stripe, 0)
    plsc.subcore_barrier()

    H = G // 2

    def grp(g, x):
        pltpu.sync_copy(src_hbm.at[s, pl.ds(g * G, G)], src_g)
        pltpu.sync_copy(dst_hbm.at[s, pl.ds(g * G, G)], dst_g)

        def fill_idx(i, y):
            r = i // (B // 16)
            k = i % (B // 16)
            v = src_g[r, pl.ds(k * 16, 16)]
            src_g[r, pl.ds(k * 16, 16)] = v + c * N
            return y

        lax.fori_loop(0, G * (B // 16), fill_idx, 0)

        descs = [None] * H
        descs[0] = pltpu.async_copy(g_hbm.at[src_g.at[c]], bufs[0], semg[0])
        for b in range(H):
            if b + 1 < H:
                descs[b + 1] = pltpu.async_copy(
                    g_hbm.at[src_g.at[2 * (b + 1) + c]], bufs[(b + 1) % 2],
                    semg[(b + 1) % 2])
            descs[b].wait()
            pltpu.sync_copy(bufs[b % 2], acc_sh.at[dst_g.at[2 * b + c]],
                            add=True)
        return x

    lax.fori_loop(0, NGRP, grp, 0)
    plsc.subcore_barrier()
    pltpu.sync_copy(acc_sh.at[pl.ds(s * STRIPE, STRIPE)],
                    acc_out.at[c, pl.ds(s * STRIPE, STRIPE)])


_agg2 = pl.kernel(
    _agg2_body,
    out_type=jax.ShapeDtypeStruct((NC, NROWS, DOUT), jnp.float32),
    mesh=_mesh,
    scratch_types=[
        pltpu.VMEM((G, B), jnp.int32),
        pltpu.VMEM((G, B), jnp.int32),
        pltpu.VMEM((B, DOUT), jnp.float32),
        pltpu.VMEM((B, DOUT), jnp.float32),
        pltpu.VMEM_SHARED((NROWS, DOUT), jnp.float32),
        pltpu.SemaphoreType.DMA,
        pltpu.SemaphoreType.DMA,
    ],
)


def _tc1_body(x_ref, w1_ref, degp_ref, g1_ref, dis_ref):
    deg = degp_ref[0, :] + degp_ref[1, :]
    dis = jnp.where(deg > 0, lax.rsqrt(deg), 0.0)
    dis_ref[...] = dis
    h = jnp.dot(x_ref[...], w1_ref[...], preferred_element_type=jnp.float32)
    g = h * dis[:N, None]
    g1_ref[0] = g[:, : DHID // 2]
    g1_ref[1] = g[:, DHID // 2:]


_tc1 = pl.pallas_call(
    _tc1_body,
    out_shape=(
        jax.ShapeDtypeStruct((2, N, DHID // 2), jnp.float32),
        jax.ShapeDtypeStruct((NROWS,), jnp.float32),
    ),
)


def _tc2_body(accp_ref, w2_ref, b1_ref, dis_ref, g2_ref):
    dis = dis_ref[...][:N]
    a = accp_ref[0, :N, :] * dis[:, None] + b1_ref[...][None, : DHID // 2]
    b = accp_ref[1, :N, :] * dis[:, None] + b1_ref[...][None, DHID // 2:]
    t = jnp.concatenate([jnp.maximum(a, 0.0), jnp.maximum(b, 0.0)], axis=1)
    g2 = jnp.dot(t, w2_ref[...], preferred_element_type=jnp.float32)
    g2_ref[...] = g2 * dis[:, None]


_tc2 = pl.pallas_call(
    _tc2_body,
    out_shape=jax.ShapeDtypeStruct((N, DOUT), jnp.float32),
)


def _tc3_body(accp_ref, b2_ref, dis_ref, out_ref):
    dis = dis_ref[...][:N]
    out = accp_ref[0, :N, :] + accp_ref[1, :N, :]
    out_ref[...] = out * dis[:, None] + b2_ref[...][None, :]


_tc3 = pl.pallas_call(
    _tc3_body,
    out_shape=jax.ShapeDtypeStruct((N, DOUT), jnp.float32),
)


def kernel(x, edge_index, W1, b1, W2, b2):
    loop = jnp.arange(N, dtype=jnp.int32)
    pad = E_PAD - E_TOT
    src = jnp.concatenate(
        [edge_index[0], loop, jnp.zeros((pad,), jnp.int32)]).reshape(NS, BLKS, B)
    dst = jnp.concatenate(
        [edge_index[1], loop, jnp.full((pad,), JUNK, jnp.int32)]).reshape(NS, BLKS, B)

    degp = _deg_kernel(dst).reshape(NC, NS * DSTRIPE)[:, :NROWS]
    g1i, dis = _tc1(x, W1, degp)
    accp1 = _agg1(g1i.reshape(2 * N, DHID // 2), src, dst)
    g2 = _tc2(accp1, W2, b1, dis)
    accp2 = _agg2(jnp.concatenate([g2, g2], axis=0), src, dst)
    return _tc3(accp2, b2, dis)
